# baseline (speedup 1.0000x reference)
import functools

import jax
import jax.numpy as jnp
from jax import lax
from jax.experimental import pallas as pl
from jax.experimental.pallas import tpu as pltpu
from jax.experimental.pallas import tpu_sc as plsc

B = 1
T = 2048
D = 768
F = 3072
E = 8
K = 2

TILE = 128
NP = T * K
P = NP + E * TILE
NT = P // TILE


def _router_body(x_ref, wg_ref, logits_ref, probs_ref, w_ref, idx_ref,
                 dst_ref, te_ref):
    x = x_ref[...]
    wg = wg_ref[...]
    logits = jnp.dot(x, wg, preferred_element_type=jnp.float32)
    m = jnp.max(logits, axis=-1, keepdims=True)
    ex = jnp.exp(logits - m)
    probs = ex / jnp.sum(ex, axis=-1, keepdims=True)
    logits_ref[...] = logits
    probs_ref[...] = probs

    cols = jax.lax.broadcasted_iota(jnp.int32, (T, E), 1)
    m1 = jnp.max(probs, axis=-1, keepdims=True)
    i1 = jnp.min(jnp.where(probs == m1, cols, E), axis=-1, keepdims=True)
    masked = jnp.where(cols == i1, -jnp.inf, probs)
    m2 = jnp.max(masked, axis=-1, keepdims=True)
    i2 = jnp.min(jnp.where(masked == m2, cols, E), axis=-1, keepdims=True)
    w_ref[:, 0:1] = m1
    w_ref[:, 1:2] = m2
    idx_ref[:, 0:1] = i1
    idx_ref[:, 1:2] = i2

    oh0 = (cols == i1)
    oh1 = (cols == i2)
    oh01 = oh0.astype(jnp.int32) + oh1.astype(jnp.int32)
    c = oh01
    s = 1
    while s < T:
        c = c + jnp.concatenate(
            [jnp.zeros((s, E), jnp.int32), c[:T - s]], axis=0)
        s *= 2
    excl = c - oh01
    counts = c[T - 1:T, :]
    padded = ((counts + (TILE - 1)) // TILE) * TILE
    tri = (jax.lax.broadcasted_iota(jnp.int32, (E, E), 0)
           <= jax.lax.broadcasted_iota(jnp.int32, (E, E), 1))
    cum_pad = jnp.dot(padded.astype(jnp.float32), tri.astype(jnp.float32),
                      preferred_element_type=jnp.float32)
    cum_pad = cum_pad.astype(jnp.int32)
    pad_off = cum_pad - padded
    dst_ref[:, 0:1] = jnp.sum(
        jnp.where(oh0, excl + pad_off, 0), axis=1, keepdims=True)
    dst_ref[:, 1:2] = jnp.sum(
        jnp.where(oh1, excl + pad_off, 0), axis=1, keepdims=True)
    starts = jax.lax.broadcasted_iota(jnp.int32, (NT, 1), 0) * TILE
    te_ref[...] = jnp.sum(
        (starts >= cum_pad).astype(jnp.int32), axis=1, keepdims=True)


def _router(x_flat, Wg):
    return pl.pallas_call(
        _router_body,
        out_shape=(
            jax.ShapeDtypeStruct((T, E), jnp.float32),
            jax.ShapeDtypeStruct((T, E), jnp.float32),
            jax.ShapeDtypeStruct((T, K), jnp.float32),
            jax.ShapeDtypeStruct((T, K), jnp.int32),
            jax.ShapeDtypeStruct((T, K), jnp.int32),
            jax.ShapeDtypeStruct((NT, 1), jnp.int32),
        ),
    )(x_flat, Wg)


def _ffn_body(te_ref, xd_ref, wp_ref, w1_ref, w3_ref, w2_ref, ys_ref):
    i = pl.program_id(0)

    @pl.when(te_ref[i] < E)
    def _():
        xb = xd_ref[...]
        a = jnp.dot(xb, w1_ref[0], preferred_element_type=jnp.float32)
        bb = jnp.dot(xb, w3_ref[0], preferred_element_type=jnp.float32)
        h = a * jax.nn.sigmoid(a) * bb
        y = jnp.dot(h, w2_ref[0], preferred_element_type=jnp.float32)
        ys_ref[...] = y * wp_ref[...]


def _expert_ffn(xd, w_of_pos, W1, W3, W2, tile_expert):
    grid_spec = pltpu.PrefetchScalarGridSpec(
        num_scalar_prefetch=1,
        grid=(NT,),
        in_specs=[
            pl.BlockSpec((TILE, D), lambda i, te: (i, 0)),
            pl.BlockSpec((TILE, 1), lambda i, te: (i, 0)),
            pl.BlockSpec((1, D, F),
                         lambda i, te: (jnp.minimum(te[i], E - 1), 0, 0)),
            pl.BlockSpec((1, D, F),
                         lambda i, te: (jnp.minimum(te[i], E - 1), 0, 0)),
            pl.BlockSpec((1, F, D),
                         lambda i, te: (jnp.minimum(te[i], E - 1), 0, 0)),
        ],
        out_specs=pl.BlockSpec((TILE, D), lambda i, te: (i, 0)),
    )
    return pl.pallas_call(
        _ffn_body,
        grid_spec=grid_spec,
        out_shape=jax.ShapeDtypeStruct((P, D), jnp.float32),
        compiler_params=pltpu.CompilerParams(
            vmem_limit_bytes=100 * 1024 * 1024),
    )(tile_expert, xd, w_of_pos, W1, W3, W2)


NC = 2
NS = 16
NW = NC * NS
TPS = T // NS
PPW = P // NW
HALF = PPW // 2
TPW = T // NW

_SC_MESH = plsc.VectorSubcoreMesh(core_axis_name="c", subcore_axis_name="s")


def _dispatch_body(dstT_hbm, wT_hbm, x_hbm, xd_hbm, wp_hbm,
                   idx0_v, idx1_v, w0_v, w1_v, rows_v, sem0):
    wid = lax.axis_index("s") * NC + lax.axis_index("c")
    tbase = wid * TPW
    cpa = pltpu.make_async_copy(x_hbm.at[pl.ds(tbase, TPW)], rows_v, sem0)
    cpa.start()
    pltpu.sync_copy(dstT_hbm.at[0, pl.ds(tbase, TPW)], idx0_v)
    pltpu.sync_copy(dstT_hbm.at[1, pl.ds(tbase, TPW)], idx1_v)
    pltpu.sync_copy(wT_hbm.at[0, pl.ds(tbase, TPW)], w0_v)
    pltpu.sync_copy(wT_hbm.at[1, pl.ds(tbase, TPW)], w1_v)
    pltpu.sync_copy(w0_v, wp_hbm.at[idx0_v])
    pltpu.sync_copy(w1_v, wp_hbm.at[idx1_v])
    cpa.wait()
    pltpu.sync_copy(rows_v, xd_hbm.at[idx0_v])
    pltpu.sync_copy(rows_v, xd_hbm.at[idx1_v])


@functools.partial(
    pl.kernel,
    out_type=(
        jax.ShapeDtypeStruct((P, D), jnp.float32),
        jax.ShapeDtypeStruct((P,), jnp.float32),
    ),
    mesh=_SC_MESH,
    scratch_types=[
        pltpu.VMEM((TPW,), jnp.int32),
        pltpu.VMEM((TPW,), jnp.int32),
        pltpu.VMEM((TPW,), jnp.float32),
        pltpu.VMEM((TPW,), jnp.float32),
        pltpu.VMEM((TPW, D), jnp.float32),
        pltpu.SemaphoreType.DMA,
    ],
)
def _sc_dispatch(dstT_hbm, wT_hbm, x_hbm, xd_hbm, wp_hbm, *rest):
    _dispatch_body(dstT_hbm, wT_hbm, x_hbm, xd_hbm, wp_hbm, *rest)


def _combine_body(ysw_hbm, dstT_hbm, out_hbm,
                  idx0_v, idx1_v, ga_v, gb_v, sem0, sem1):
    wid = lax.axis_index("s") * NC + lax.axis_index("c")
    tbase = wid * TPW
    pltpu.sync_copy(dstT_hbm.at[0, pl.ds(tbase, TPW)], idx0_v)
    pltpu.sync_copy(dstT_hbm.at[1, pl.ds(tbase, TPW)], idx1_v)
    cpa = pltpu.make_async_copy(ysw_hbm.at[idx0_v], ga_v, sem0)
    cpb = pltpu.make_async_copy(ysw_hbm.at[idx1_v], gb_v, sem1)
    cpa.start()
    cpb.start()
    cpa.wait()
    cpb.wait()

    def row(r, carry):
        for c in range(D // 16):
            sl = pl.ds(c * 16, 16)
            ga_v[r, sl] = ga_v[r, sl] + gb_v[r, sl]
        return carry

    lax.fori_loop(0, TPW, row, 0)
    pltpu.sync_copy(ga_v, out_hbm.at[pl.ds(tbase, TPW)])


@functools.partial(
    pl.kernel,
    out_type=jax.ShapeDtypeStruct((T, D), jnp.float32),
    mesh=_SC_MESH,
    scratch_types=[
        pltpu.VMEM((TPW,), jnp.int32),
        pltpu.VMEM((TPW,), jnp.int32),
        pltpu.VMEM((TPW, D), jnp.float32),
        pltpu.VMEM((TPW, D), jnp.float32),
        pltpu.SemaphoreType.DMA,
        pltpu.SemaphoreType.DMA,
    ],
)
def _sc_combine(ysw_hbm, dstT_hbm, out_hbm, *rest):
    _combine_body(ysw_hbm, dstT_hbm, out_hbm, *rest)


def kernel(x, Wg, W1, W3, W2):
    x_flat = x.reshape(T, D)
    logits, probs, topk_w, topk_idx, dst, te = _router(x_flat, Wg)

    dstT = dst.T
    wT = topk_w.T
    xd, w_of_pos = _sc_dispatch(dstT, wT, x_flat)
    ys = _expert_ffn(xd, w_of_pos.reshape(P, 1), W1, W3, W2,
                     te.reshape(NT))
    out = _sc_combine(ys, dstT)
    return out.reshape(B, T, D), probs, logits, topk_idx


# speedup vs baseline: 1.0658x; 1.0658x over previous
import functools

import jax
import jax.numpy as jnp
from jax import lax
from jax.experimental import pallas as pl
from jax.experimental.pallas import tpu as pltpu
from jax.experimental.pallas import tpu_sc as plsc

B = 1
T = 2048
D = 768
F = 3072
E = 8
K = 2

TILE = 128
NP = T * K
P = NP + E * TILE
NT = P // TILE


def _router_body(x_ref, wg_ref, logits_ref, probs_ref, w_ref, idx_ref,
                 dst_ref, te_ref):
    x = x_ref[...]
    wg = wg_ref[...]
    logits = jnp.dot(x, wg, preferred_element_type=jnp.float32)
    m = jnp.max(logits, axis=-1, keepdims=True)
    ex = jnp.exp(logits - m)
    probs = ex / jnp.sum(ex, axis=-1, keepdims=True)
    logits_ref[...] = logits
    probs_ref[...] = probs

    cols = jax.lax.broadcasted_iota(jnp.int32, (T, E), 1)
    m1 = jnp.max(probs, axis=-1, keepdims=True)
    i1 = jnp.min(jnp.where(probs == m1, cols, E), axis=-1, keepdims=True)
    masked = jnp.where(cols == i1, -jnp.inf, probs)
    m2 = jnp.max(masked, axis=-1, keepdims=True)
    i2 = jnp.min(jnp.where(masked == m2, cols, E), axis=-1, keepdims=True)
    w_ref[:, 0:1] = m1
    w_ref[:, 1:2] = m2
    idx_ref[:, 0:1] = i1
    idx_ref[:, 1:2] = i2

    oh0 = (cols == i1)
    oh1 = (cols == i2)
    oh01 = oh0.astype(jnp.int32) + oh1.astype(jnp.int32)
    c = oh01
    s = 1
    while s < T:
        c = c + jnp.concatenate(
            [jnp.zeros((s, E), jnp.int32), c[:T - s]], axis=0)
        s *= 2
    excl = c - oh01
    counts = c[T - 1:T, :]
    padded = ((counts + (TILE - 1)) // TILE) * TILE
    tri = (jax.lax.broadcasted_iota(jnp.int32, (E, E), 0)
           <= jax.lax.broadcasted_iota(jnp.int32, (E, E), 1))
    cum_pad = jnp.dot(padded.astype(jnp.float32), tri.astype(jnp.float32),
                      preferred_element_type=jnp.float32)
    cum_pad = cum_pad.astype(jnp.int32)
    pad_off = cum_pad - padded
    dst_ref[:, 0:1] = jnp.sum(
        jnp.where(oh0, excl + pad_off, 0), axis=1, keepdims=True)
    dst_ref[:, 1:2] = jnp.sum(
        jnp.where(oh1, excl + pad_off, 0), axis=1, keepdims=True)
    starts = jax.lax.broadcasted_iota(jnp.int32, (NT, 1), 0) * TILE
    te_ref[...] = jnp.sum(
        (starts >= cum_pad).astype(jnp.int32), axis=1, keepdims=True)


def _router(x_flat, Wg):
    return pl.pallas_call(
        _router_body,
        out_shape=(
            jax.ShapeDtypeStruct((T, E), jnp.float32),
            jax.ShapeDtypeStruct((T, E), jnp.float32),
            jax.ShapeDtypeStruct((T, K), jnp.float32),
            jax.ShapeDtypeStruct((T, K), jnp.int32),
            jax.ShapeDtypeStruct((T, K), jnp.int32),
            jax.ShapeDtypeStruct((NT, 1), jnp.int32),
        ),
    )(x_flat, Wg)


def _weight_copies(w1_hbm, w3_hbm, w2_hbm, w1b, w3b, w2b, sems, e, slot):
    return (
        pltpu.make_async_copy(w1_hbm.at[e], w1b.at[slot], sems.at[slot, 0]),
        pltpu.make_async_copy(w3_hbm.at[e], w3b.at[slot], sems.at[slot, 1]),
        pltpu.make_async_copy(w2_hbm.at[e], w2b.at[slot], sems.at[slot, 2]),
    )


def _ffn_body(te_ref, first_ref, slot_ref, nxt_ref,
              xd_ref, wp_ref, w1_hbm, w3_hbm, w2_hbm, ys_ref,
              w1b, w3b, w2b, sems):
    i = pl.program_id(0)
    s = slot_ref[i]

    @pl.when(i == 0)
    def _():
        for cp in _weight_copies(w1_hbm, w3_hbm, w2_hbm,
                                 w1b, w3b, w2b, sems, te_ref[0], 0):
            cp.start()

    @pl.when((first_ref[i] == 1) & (te_ref[i] < E))
    def _():
        @pl.when(nxt_ref[i] >= 0)
        def _():
            for cp in _weight_copies(w1_hbm, w3_hbm, w2_hbm,
                                     w1b, w3b, w2b, sems,
                                     nxt_ref[i], 1 - s):
                cp.start()

        for cp in _weight_copies(w1_hbm, w3_hbm, w2_hbm,
                                 w1b, w3b, w2b, sems, te_ref[i], s):
            cp.wait()

    @pl.when(te_ref[i] < E)
    def _():
        xb = xd_ref[...]
        a = jnp.dot(xb, w1b[s], preferred_element_type=jnp.float32)
        bb = jnp.dot(xb, w3b[s], preferred_element_type=jnp.float32)
        h = a * jax.nn.sigmoid(a) * bb
        y = jnp.dot(h, w2b[s], preferred_element_type=jnp.float32)
        ys_ref[...] = y * wp_ref[...]


def _expert_ffn(xd, w_of_pos, W1, W3, W2, tile_expert):
    te = tile_expert
    first = jnp.concatenate(
        [jnp.ones((1,), jnp.int32), (te[1:] != te[:-1]).astype(jnp.int32)])
    slot = (jnp.cumsum(first) - 1) % 2
    nxt = jnp.min(jnp.where(te[None, :] > te[:, None], te[None, :], E),
                  axis=1)
    nxt = jnp.where((nxt < E) & (te < E), nxt, -1)

    grid_spec = pltpu.PrefetchScalarGridSpec(
        num_scalar_prefetch=4,
        grid=(NT,),
        in_specs=[
            pl.BlockSpec((TILE, D), lambda i, *_: (i, 0)),
            pl.BlockSpec((TILE, 1), lambda i, *_: (i, 0)),
            pl.BlockSpec(memory_space=pl.ANY),
            pl.BlockSpec(memory_space=pl.ANY),
            pl.BlockSpec(memory_space=pl.ANY),
        ],
        out_specs=pl.BlockSpec((TILE, D), lambda i, *_: (i, 0)),
        scratch_shapes=[
            pltpu.VMEM((2, D, F), jnp.float32),
            pltpu.VMEM((2, D, F), jnp.float32),
            pltpu.VMEM((2, F, D), jnp.float32),
            pltpu.SemaphoreType.DMA((2, 3)),
        ],
    )
    return pl.pallas_call(
        _ffn_body,
        grid_spec=grid_spec,
        out_shape=jax.ShapeDtypeStruct((P, D), jnp.float32),
        compiler_params=pltpu.CompilerParams(
            vmem_limit_bytes=100 * 1024 * 1024),
    )(te, first, slot, nxt, xd, w_of_pos, W1, W3, W2)


NC = 2
NS = 16
NW = NC * NS
TPS = T // NS
PPW = P // NW
HALF = PPW // 2
TPW = T // NW

_SC_MESH = plsc.VectorSubcoreMesh(core_axis_name="c", subcore_axis_name="s")


def _dispatch_body(dstT_hbm, wT_hbm, x_hbm, xd_hbm, wp_hbm,
                   idx0_v, idx1_v, w0_v, w1_v, rows_v, sem0):
    wid = lax.axis_index("s") * NC + lax.axis_index("c")
    tbase = wid * TPW
    cpa = pltpu.make_async_copy(x_hbm.at[pl.ds(tbase, TPW)], rows_v, sem0)
    cpa.start()
    pltpu.sync_copy(dstT_hbm.at[0, pl.ds(tbase, TPW)], idx0_v)
    pltpu.sync_copy(dstT_hbm.at[1, pl.ds(tbase, TPW)], idx1_v)
    pltpu.sync_copy(wT_hbm.at[0, pl.ds(tbase, TPW)], w0_v)
    pltpu.sync_copy(wT_hbm.at[1, pl.ds(tbase, TPW)], w1_v)
    pltpu.sync_copy(w0_v, wp_hbm.at[idx0_v])
    pltpu.sync_copy(w1_v, wp_hbm.at[idx1_v])
    cpa.wait()
    pltpu.sync_copy(rows_v, xd_hbm.at[idx0_v])
    pltpu.sync_copy(rows_v, xd_hbm.at[idx1_v])


@functools.partial(
    pl.kernel,
    out_type=(
        jax.ShapeDtypeStruct((P, D), jnp.float32),
        jax.ShapeDtypeStruct((P,), jnp.float32),
    ),
    mesh=_SC_MESH,
    scratch_types=[
        pltpu.VMEM((TPW,), jnp.int32),
        pltpu.VMEM((TPW,), jnp.int32),
        pltpu.VMEM((TPW,), jnp.float32),
        pltpu.VMEM((TPW,), jnp.float32),
        pltpu.VMEM((TPW, D), jnp.float32),
        pltpu.SemaphoreType.DMA,
    ],
)
def _sc_dispatch(dstT_hbm, wT_hbm, x_hbm, xd_hbm, wp_hbm, *rest):
    _dispatch_body(dstT_hbm, wT_hbm, x_hbm, xd_hbm, wp_hbm, *rest)


def _combine_body(ysw_hbm, dstT_hbm, out_hbm,
                  idx0_v, idx1_v, ga_v, gb_v, sem0, sem1):
    wid = lax.axis_index("s") * NC + lax.axis_index("c")
    tbase = wid * TPW
    pltpu.sync_copy(dstT_hbm.at[0, pl.ds(tbase, TPW)], idx0_v)
    pltpu.sync_copy(dstT_hbm.at[1, pl.ds(tbase, TPW)], idx1_v)
    cpa = pltpu.make_async_copy(ysw_hbm.at[idx0_v], ga_v, sem0)
    cpb = pltpu.make_async_copy(ysw_hbm.at[idx1_v], gb_v, sem1)
    cpa.start()
    cpb.start()
    cpa.wait()
    cpb.wait()

    def row(r, carry):
        for c in range(D // 16):
            sl = pl.ds(c * 16, 16)
            ga_v[r, sl] = ga_v[r, sl] + gb_v[r, sl]
        return carry

    lax.fori_loop(0, TPW, row, 0)
    pltpu.sync_copy(ga_v, out_hbm.at[pl.ds(tbase, TPW)])


@functools.partial(
    pl.kernel,
    out_type=jax.ShapeDtypeStruct((T, D), jnp.float32),
    mesh=_SC_MESH,
    scratch_types=[
        pltpu.VMEM((TPW,), jnp.int32),
        pltpu.VMEM((TPW,), jnp.int32),
        pltpu.VMEM((TPW, D), jnp.float32),
        pltpu.VMEM((TPW, D), jnp.float32),
        pltpu.SemaphoreType.DMA,
        pltpu.SemaphoreType.DMA,
    ],
)
def _sc_combine(ysw_hbm, dstT_hbm, out_hbm, *rest):
    _combine_body(ysw_hbm, dstT_hbm, out_hbm, *rest)


def kernel(x, Wg, W1, W3, W2):
    x_flat = x.reshape(T, D)
    logits, probs, topk_w, topk_idx, dst, te = _router(x_flat, Wg)

    dstT = dst.T
    wT = topk_w.T
    xd, w_of_pos = _sc_dispatch(dstT, wT, x_flat)
    ys = _expert_ffn(xd, w_of_pos.reshape(P, 1), W1, W3, W2,
                     te.reshape(NT))
    out = _sc_combine(ys, dstT)
    return out.reshape(B, T, D), probs, logits, topk_idx


# speedup vs baseline: 1.1902x; 1.1168x over previous
import functools

import jax
import jax.numpy as jnp
from jax import lax
from jax.experimental import pallas as pl
from jax.experimental.pallas import tpu as pltpu
from jax.experimental.pallas import tpu_sc as plsc

B = 1
T = 2048
D = 768
F = 3072
E = 8
K = 2

TILE = 256
NP = T * K
P = NP + E * TILE
NT = P // TILE


def _router_body(x_ref, wg_ref, logits_ref, probs_ref, w_ref, idx_ref,
                 dst_ref, te_ref):
    x = x_ref[...]
    wg = wg_ref[...]
    logits = jnp.dot(x, wg, preferred_element_type=jnp.float32)
    m = jnp.max(logits, axis=-1, keepdims=True)
    ex = jnp.exp(logits - m)
    probs = ex / jnp.sum(ex, axis=-1, keepdims=True)
    logits_ref[...] = logits
    probs_ref[...] = probs

    cols = jax.lax.broadcasted_iota(jnp.int32, (T, E), 1)
    m1 = jnp.max(probs, axis=-1, keepdims=True)
    i1 = jnp.min(jnp.where(probs == m1, cols, E), axis=-1, keepdims=True)
    masked = jnp.where(cols == i1, -jnp.inf, probs)
    m2 = jnp.max(masked, axis=-1, keepdims=True)
    i2 = jnp.min(jnp.where(masked == m2, cols, E), axis=-1, keepdims=True)
    w_ref[:, 0:1] = m1
    w_ref[:, 1:2] = m2
    idx_ref[:, 0:1] = i1
    idx_ref[:, 1:2] = i2

    oh0 = (cols == i1)
    oh1 = (cols == i2)
    oh01 = oh0.astype(jnp.int32) + oh1.astype(jnp.int32)
    c = oh01
    s = 1
    while s < T:
        c = c + jnp.concatenate(
            [jnp.zeros((s, E), jnp.int32), c[:T - s]], axis=0)
        s *= 2
    excl = c - oh01
    counts = c[T - 1:T, :]
    padded = ((counts + (TILE - 1)) // TILE) * TILE
    tri = (jax.lax.broadcasted_iota(jnp.int32, (E, E), 0)
           <= jax.lax.broadcasted_iota(jnp.int32, (E, E), 1))
    cum_pad = jnp.dot(padded.astype(jnp.float32), tri.astype(jnp.float32),
                      preferred_element_type=jnp.float32)
    cum_pad = cum_pad.astype(jnp.int32)
    pad_off = cum_pad - padded
    dst_ref[:, 0:1] = jnp.sum(
        jnp.where(oh0, excl + pad_off, 0), axis=1, keepdims=True)
    dst_ref[:, 1:2] = jnp.sum(
        jnp.where(oh1, excl + pad_off, 0), axis=1, keepdims=True)
    starts = jax.lax.broadcasted_iota(jnp.int32, (NT, 1), 0) * TILE
    te_ref[...] = jnp.sum(
        (starts >= cum_pad).astype(jnp.int32), axis=1, keepdims=True)


def _router(x_flat, Wg):
    return pl.pallas_call(
        _router_body,
        out_shape=(
            jax.ShapeDtypeStruct((T, E), jnp.float32),
            jax.ShapeDtypeStruct((T, E), jnp.float32),
            jax.ShapeDtypeStruct((T, K), jnp.float32),
            jax.ShapeDtypeStruct((T, K), jnp.int32),
            jax.ShapeDtypeStruct((T, K), jnp.int32),
            jax.ShapeDtypeStruct((NT, 1), jnp.int32),
        ),
    )(x_flat, Wg)


def _weight_copies(w1_hbm, w3_hbm, w2_hbm, w1b, w3b, w2b, sems, e, slot):
    return (
        pltpu.make_async_copy(w1_hbm.at[e], w1b.at[slot], sems.at[slot, 0]),
        pltpu.make_async_copy(w3_hbm.at[e], w3b.at[slot], sems.at[slot, 1]),
        pltpu.make_async_copy(w2_hbm.at[e], w2b.at[slot], sems.at[slot, 2]),
    )


def _ffn_body(te_ref, first_ref, slot_ref, nxt_ref,
              xd_ref, wp_ref, w1_hbm, w3_hbm, w2_hbm, ys_ref,
              w1b, w3b, w2b, sems):
    i = pl.program_id(0)
    s = slot_ref[i]

    @pl.when(i == 0)
    def _():
        for cp in _weight_copies(w1_hbm, w3_hbm, w2_hbm,
                                 w1b, w3b, w2b, sems, te_ref[0], 0):
            cp.start()

    @pl.when((first_ref[i] == 1) & (te_ref[i] < E))
    def _():
        @pl.when(nxt_ref[i] >= 0)
        def _():
            for cp in _weight_copies(w1_hbm, w3_hbm, w2_hbm,
                                     w1b, w3b, w2b, sems,
                                     nxt_ref[i], 1 - s):
                cp.start()

        for cp in _weight_copies(w1_hbm, w3_hbm, w2_hbm,
                                 w1b, w3b, w2b, sems, te_ref[i], s):
            cp.wait()

    @pl.when(te_ref[i] < E)
    def _():
        xb = xd_ref[...]
        a = jnp.dot(xb, w1b[s], preferred_element_type=jnp.float32)
        bb = jnp.dot(xb, w3b[s], preferred_element_type=jnp.float32)
        h = a * jax.nn.sigmoid(a) * bb
        y = jnp.dot(h, w2b[s], preferred_element_type=jnp.float32)
        ys_ref[...] = y * wp_ref[...]


def _expert_ffn(xd, w_of_pos, W1, W3, W2, tile_expert):
    te = tile_expert
    first = jnp.concatenate(
        [jnp.ones((1,), jnp.int32), (te[1:] != te[:-1]).astype(jnp.int32)])
    slot = (jnp.cumsum(first) - 1) % 2
    nxt = jnp.min(jnp.where(te[None, :] > te[:, None], te[None, :], E),
                  axis=1)
    nxt = jnp.where((nxt < E) & (te < E), nxt, -1)

    grid_spec = pltpu.PrefetchScalarGridSpec(
        num_scalar_prefetch=4,
        grid=(NT,),
        in_specs=[
            pl.BlockSpec((TILE, D), lambda i, *_: (i, 0)),
            pl.BlockSpec((TILE, 1), lambda i, *_: (i, 0)),
            pl.BlockSpec(memory_space=pl.ANY),
            pl.BlockSpec(memory_space=pl.ANY),
            pl.BlockSpec(memory_space=pl.ANY),
        ],
        out_specs=pl.BlockSpec((TILE, D), lambda i, *_: (i, 0)),
        scratch_shapes=[
            pltpu.VMEM((2, D, F), jnp.float32),
            pltpu.VMEM((2, D, F), jnp.float32),
            pltpu.VMEM((2, F, D), jnp.float32),
            pltpu.SemaphoreType.DMA((2, 3)),
        ],
    )
    return pl.pallas_call(
        _ffn_body,
        grid_spec=grid_spec,
        out_shape=jax.ShapeDtypeStruct((P, D), jnp.float32),
        compiler_params=pltpu.CompilerParams(
            vmem_limit_bytes=100 * 1024 * 1024),
    )(te, first, slot, nxt, xd, w_of_pos, W1, W3, W2)


NC = 2
NS = 16
NW = NC * NS
TPS = T // NS
PPW = P // NW
HALF = PPW // 2
TPW = T // NW

_SC_MESH = plsc.VectorSubcoreMesh(core_axis_name="c", subcore_axis_name="s")


def _dispatch_body(dstT_hbm, wT_hbm, x_hbm, xd_hbm, wp_hbm,
                   idx0_v, idx1_v, w0_v, w1_v, rows_v, sem0):
    wid = lax.axis_index("s") * NC + lax.axis_index("c")
    tbase = wid * TPW
    cpa = pltpu.make_async_copy(x_hbm.at[pl.ds(tbase, TPW)], rows_v, sem0)
    cpa.start()
    pltpu.sync_copy(dstT_hbm.at[0, pl.ds(tbase, TPW)], idx0_v)
    pltpu.sync_copy(dstT_hbm.at[1, pl.ds(tbase, TPW)], idx1_v)
    pltpu.sync_copy(wT_hbm.at[0, pl.ds(tbase, TPW)], w0_v)
    pltpu.sync_copy(wT_hbm.at[1, pl.ds(tbase, TPW)], w1_v)
    pltpu.sync_copy(w0_v, wp_hbm.at[idx0_v])
    pltpu.sync_copy(w1_v, wp_hbm.at[idx1_v])
    cpa.wait()
    pltpu.sync_copy(rows_v, xd_hbm.at[idx0_v])
    pltpu.sync_copy(rows_v, xd_hbm.at[idx1_v])


@functools.partial(
    pl.kernel,
    out_type=(
        jax.ShapeDtypeStruct((P, D), jnp.float32),
        jax.ShapeDtypeStruct((P,), jnp.float32),
    ),
    mesh=_SC_MESH,
    scratch_types=[
        pltpu.VMEM((TPW,), jnp.int32),
        pltpu.VMEM((TPW,), jnp.int32),
        pltpu.VMEM((TPW,), jnp.float32),
        pltpu.VMEM((TPW,), jnp.float32),
        pltpu.VMEM((TPW, D), jnp.float32),
        pltpu.SemaphoreType.DMA,
    ],
)
def _sc_dispatch(dstT_hbm, wT_hbm, x_hbm, xd_hbm, wp_hbm, *rest):
    _dispatch_body(dstT_hbm, wT_hbm, x_hbm, xd_hbm, wp_hbm, *rest)


def _combine_body(ysw_hbm, dstT_hbm, out_hbm,
                  idx0_v, idx1_v, ga_v, gb_v, sem0, sem1):
    wid = lax.axis_index("s") * NC + lax.axis_index("c")
    tbase = wid * TPW
    pltpu.sync_copy(dstT_hbm.at[0, pl.ds(tbase, TPW)], idx0_v)
    pltpu.sync_copy(dstT_hbm.at[1, pl.ds(tbase, TPW)], idx1_v)
    cpa = pltpu.make_async_copy(ysw_hbm.at[idx0_v], ga_v, sem0)
    cpb = pltpu.make_async_copy(ysw_hbm.at[idx1_v], gb_v, sem1)
    cpa.start()
    cpb.start()
    cpa.wait()
    cpb.wait()

    def row(r, carry):
        for c in range(D // 16):
            sl = pl.ds(c * 16, 16)
            ga_v[r, sl] = ga_v[r, sl] + gb_v[r, sl]
        return carry

    lax.fori_loop(0, TPW, row, 0)
    pltpu.sync_copy(ga_v, out_hbm.at[pl.ds(tbase, TPW)])


@functools.partial(
    pl.kernel,
    out_type=jax.ShapeDtypeStruct((T, D), jnp.float32),
    mesh=_SC_MESH,
    scratch_types=[
        pltpu.VMEM((TPW,), jnp.int32),
        pltpu.VMEM((TPW,), jnp.int32),
        pltpu.VMEM((TPW, D), jnp.float32),
        pltpu.VMEM((TPW, D), jnp.float32),
        pltpu.SemaphoreType.DMA,
        pltpu.SemaphoreType.DMA,
    ],
)
def _sc_combine(ysw_hbm, dstT_hbm, out_hbm, *rest):
    _combine_body(ysw_hbm, dstT_hbm, out_hbm, *rest)


def kernel(x, Wg, W1, W3, W2):
    x_flat = x.reshape(T, D)
    logits, probs, topk_w, topk_idx, dst, te = _router(x_flat, Wg)

    dstT = dst.T
    wT = topk_w.T
    xd, w_of_pos = _sc_dispatch(dstT, wT, x_flat)
    ys = _expert_ffn(xd, w_of_pos.reshape(P, 1), W1, W3, W2,
                     te.reshape(NT))
    out = _sc_combine(ys, dstT)
    return out.reshape(B, T, D), probs, logits, topk_idx


# speedup vs baseline: 1.2140x; 1.0200x over previous
import functools

import jax
import jax.numpy as jnp
from jax import lax
from jax.experimental import pallas as pl
from jax.experimental.pallas import tpu as pltpu
from jax.experimental.pallas import tpu_sc as plsc

B = 1
T = 2048
D = 768
F = 3072
E = 8
K = 2

TILE = 256
NP = T * K
P = NP + E * TILE
NT = P // TILE


def _router_body(x_ref, wg_ref, logits_ref, probs_ref, w_ref, idx_ref,
                 dst_ref, te_ref):
    x = x_ref[...]
    wg = wg_ref[...]
    logits = jnp.dot(x, wg, preferred_element_type=jnp.float32)
    m = jnp.max(logits, axis=-1, keepdims=True)
    ex = jnp.exp(logits - m)
    probs = ex / jnp.sum(ex, axis=-1, keepdims=True)
    logits_ref[...] = logits
    probs_ref[...] = probs

    cols = jax.lax.broadcasted_iota(jnp.int32, (T, E), 1)
    m1 = jnp.max(probs, axis=-1, keepdims=True)
    i1 = jnp.min(jnp.where(probs == m1, cols, E), axis=-1, keepdims=True)
    masked = jnp.where(cols == i1, -jnp.inf, probs)
    m2 = jnp.max(masked, axis=-1, keepdims=True)
    i2 = jnp.min(jnp.where(masked == m2, cols, E), axis=-1, keepdims=True)
    w_ref[:, 0:1] = m1
    w_ref[:, 1:2] = m2
    idx_ref[:, 0:1] = i1
    idx_ref[:, 1:2] = i2

    oh0 = (cols == i1)
    oh1 = (cols == i2)
    oh01 = oh0.astype(jnp.int32) + oh1.astype(jnp.int32)
    c = oh01
    s = 1
    while s < T:
        c = c + jnp.concatenate(
            [jnp.zeros((s, E), jnp.int32), c[:T - s]], axis=0)
        s *= 2
    excl = c - oh01
    counts = c[T - 1:T, :]
    padded = ((counts + (TILE - 1)) // TILE) * TILE
    tri = (jax.lax.broadcasted_iota(jnp.int32, (E, E), 0)
           <= jax.lax.broadcasted_iota(jnp.int32, (E, E), 1))
    cum_pad = jnp.dot(padded.astype(jnp.float32), tri.astype(jnp.float32),
                      preferred_element_type=jnp.float32)
    cum_pad = cum_pad.astype(jnp.int32)
    pad_off = cum_pad - padded
    dst_ref[:, 0:1] = jnp.sum(
        jnp.where(oh0, excl + pad_off, 0), axis=1, keepdims=True)
    dst_ref[:, 1:2] = jnp.sum(
        jnp.where(oh1, excl + pad_off, 0), axis=1, keepdims=True)
    starts = jax.lax.broadcasted_iota(jnp.int32, (NT, 1), 0) * TILE
    te_ref[...] = jnp.sum(
        (starts >= cum_pad).astype(jnp.int32), axis=1, keepdims=True)


def _router(x_flat, Wg):
    return pl.pallas_call(
        _router_body,
        out_shape=(
            jax.ShapeDtypeStruct((T, E), jnp.float32),
            jax.ShapeDtypeStruct((T, E), jnp.float32),
            jax.ShapeDtypeStruct((T, K), jnp.float32),
            jax.ShapeDtypeStruct((T, K), jnp.int32),
            jax.ShapeDtypeStruct((T, K), jnp.int32),
            jax.ShapeDtypeStruct((NT, 1), jnp.int32),
        ),
    )(x_flat, Wg)


def _weight_copies(w1_hbm, w3_hbm, w2_hbm, w1b, w3b, w2b, sems, e, slot):
    return (
        pltpu.make_async_copy(w1_hbm.at[e], w1b.at[slot], sems.at[slot, 0]),
        pltpu.make_async_copy(w3_hbm.at[e], w3b.at[slot], sems.at[slot, 1]),
        pltpu.make_async_copy(w2_hbm.at[e], w2b.at[slot], sems.at[slot, 2]),
    )


def _ffn_body(te_ref, first_ref, slot_ref, nxt_ref,
              xd_ref, wp_ref, w1_hbm, w3_hbm, w2_hbm, ys_ref,
              w1b, w3b, w2b, sems):
    i = pl.program_id(0)
    s = slot_ref[i]

    @pl.when(i == 0)
    def _():
        for cp in _weight_copies(w1_hbm, w3_hbm, w2_hbm,
                                 w1b, w3b, w2b, sems, te_ref[0], 0):
            cp.start()

    @pl.when((first_ref[i] == 1) & (te_ref[i] < E))
    def _():
        @pl.when(nxt_ref[i] >= 0)
        def _():
            for cp in _weight_copies(w1_hbm, w3_hbm, w2_hbm,
                                     w1b, w3b, w2b, sems,
                                     nxt_ref[i], 1 - s):
                cp.start()

        for cp in _weight_copies(w1_hbm, w3_hbm, w2_hbm,
                                 w1b, w3b, w2b, sems, te_ref[i], s):
            cp.wait()

    @pl.when(te_ref[i] < E)
    def _():
        xb = xd_ref[...]
        a = jnp.dot(xb, w1b[s], preferred_element_type=jnp.float32)
        bb = jnp.dot(xb, w3b[s], preferred_element_type=jnp.float32)
        h = a * jax.nn.sigmoid(a) * bb
        y = jnp.dot(h, w2b[s], preferred_element_type=jnp.float32)
        ys_ref[...] = y * wp_ref[...]


def _expert_ffn(xd, w_of_pos, W1, W3, W2, tile_expert):
    te = tile_expert
    first = jnp.concatenate(
        [jnp.ones((1,), jnp.int32), (te[1:] != te[:-1]).astype(jnp.int32)])
    slot = (jnp.cumsum(first) - 1) % 2
    nxt = jnp.min(jnp.where(te[None, :] > te[:, None], te[None, :], E),
                  axis=1)
    nxt = jnp.where((nxt < E) & (te < E), nxt, -1)

    grid_spec = pltpu.PrefetchScalarGridSpec(
        num_scalar_prefetch=4,
        grid=(NT,),
        in_specs=[
            pl.BlockSpec((TILE, D), lambda i, *_: (i, 0)),
            pl.BlockSpec((TILE, 1), lambda i, *_: (i, 0)),
            pl.BlockSpec(memory_space=pl.ANY),
            pl.BlockSpec(memory_space=pl.ANY),
            pl.BlockSpec(memory_space=pl.ANY),
        ],
        out_specs=pl.BlockSpec((TILE, D), lambda i, *_: (i, 0)),
        scratch_shapes=[
            pltpu.VMEM((2, D, F), jnp.float32),
            pltpu.VMEM((2, D, F), jnp.float32),
            pltpu.VMEM((2, F, D), jnp.float32),
            pltpu.SemaphoreType.DMA((2, 3)),
        ],
    )
    return pl.pallas_call(
        _ffn_body,
        grid_spec=grid_spec,
        out_shape=jax.ShapeDtypeStruct((P, D), jnp.float32),
        compiler_params=pltpu.CompilerParams(
            vmem_limit_bytes=100 * 1024 * 1024),
    )(te, first, slot, nxt, xd, w_of_pos, W1, W3, W2)


NC = 2
NS = 16
NW = NC * NS
TPS = T // NS
PPW = P // NW
HALF = PPW // 2
TPW = T // NW

_SC_MESH = plsc.VectorSubcoreMesh(core_axis_name="c", subcore_axis_name="s")


def _dispatch_body(dstT_hbm, wT_hbm, x_hbm, xd_hbm, wp_hbm,
                   idx0_v, idx1_v, w0_v, w1_v, rows_v, sems):
    wid = lax.axis_index("s") * NC + lax.axis_index("c")
    tbase = wid * TPW
    sl = pl.ds(tbase, TPW)
    cpx = pltpu.make_async_copy(x_hbm.at[sl], rows_v, sems.at[0])
    c0 = pltpu.make_async_copy(dstT_hbm.at[0, sl], idx0_v, sems.at[1])
    c1 = pltpu.make_async_copy(dstT_hbm.at[1, sl], idx1_v, sems.at[2])
    c2 = pltpu.make_async_copy(wT_hbm.at[0, sl], w0_v, sems.at[3])
    c3 = pltpu.make_async_copy(wT_hbm.at[1, sl], w1_v, sems.at[4])
    for cp in (cpx, c0, c1, c2, c3):
        cp.start()
    c0.wait()
    c1.wait()
    c2.wait()
    c3.wait()
    s0 = pltpu.make_async_copy(w0_v, wp_hbm.at[idx0_v], sems.at[1])
    s1 = pltpu.make_async_copy(w1_v, wp_hbm.at[idx1_v], sems.at[2])
    s0.start()
    s1.start()
    cpx.wait()
    s2 = pltpu.make_async_copy(rows_v, xd_hbm.at[idx0_v], sems.at[3])
    s3 = pltpu.make_async_copy(rows_v, xd_hbm.at[idx1_v], sems.at[4])
    s2.start()
    s3.start()
    for cp in (s0, s1, s2, s3):
        cp.wait()


@functools.partial(
    pl.kernel,
    out_type=(
        jax.ShapeDtypeStruct((P, D), jnp.float32),
        jax.ShapeDtypeStruct((P,), jnp.float32),
    ),
    mesh=_SC_MESH,
    scratch_types=[
        pltpu.VMEM((TPW,), jnp.int32),
        pltpu.VMEM((TPW,), jnp.int32),
        pltpu.VMEM((TPW,), jnp.float32),
        pltpu.VMEM((TPW,), jnp.float32),
        pltpu.VMEM((TPW, D), jnp.float32),
        pltpu.SemaphoreType.DMA((5,)),
    ],
)
def _sc_dispatch(dstT_hbm, wT_hbm, x_hbm, xd_hbm, wp_hbm, *rest):
    _dispatch_body(dstT_hbm, wT_hbm, x_hbm, xd_hbm, wp_hbm, *rest)


def _combine_body(ysw_hbm, dstT_hbm, out_hbm,
                  idx0_v, idx1_v, ga_v, gb_v, sem0, sem1):
    wid = lax.axis_index("s") * NC + lax.axis_index("c")
    tbase = wid * TPW
    pltpu.sync_copy(dstT_hbm.at[0, pl.ds(tbase, TPW)], idx0_v)
    pltpu.sync_copy(dstT_hbm.at[1, pl.ds(tbase, TPW)], idx1_v)
    cpa = pltpu.make_async_copy(ysw_hbm.at[idx0_v], ga_v, sem0)
    cpb = pltpu.make_async_copy(ysw_hbm.at[idx1_v], gb_v, sem1)
    cpa.start()
    cpb.start()
    cpa.wait()
    cpb.wait()

    def row(r, carry):
        for c in range(D // 16):
            sl = pl.ds(c * 16, 16)
            ga_v[r, sl] = ga_v[r, sl] + gb_v[r, sl]
        return carry

    lax.fori_loop(0, TPW, row, 0)
    pltpu.sync_copy(ga_v, out_hbm.at[pl.ds(tbase, TPW)])


@functools.partial(
    pl.kernel,
    out_type=jax.ShapeDtypeStruct((T, D), jnp.float32),
    mesh=_SC_MESH,
    scratch_types=[
        pltpu.VMEM((TPW,), jnp.int32),
        pltpu.VMEM((TPW,), jnp.int32),
        pltpu.VMEM((TPW, D), jnp.float32),
        pltpu.VMEM((TPW, D), jnp.float32),
        pltpu.SemaphoreType.DMA,
        pltpu.SemaphoreType.DMA,
    ],
)
def _sc_combine(ysw_hbm, dstT_hbm, out_hbm, *rest):
    _combine_body(ysw_hbm, dstT_hbm, out_hbm, *rest)


def kernel(x, Wg, W1, W3, W2):
    x_flat = x.reshape(T, D)
    logits, probs, topk_w, topk_idx, dst, te = _router(x_flat, Wg)

    dstT = dst.T
    wT = topk_w.T
    xd, w_of_pos = _sc_dispatch(dstT, wT, x_flat)
    ys = _expert_ffn(xd, w_of_pos.reshape(P, 1), W1, W3, W2,
                     te.reshape(NT))
    out = _sc_combine(ys, dstT)
    return out.reshape(B, T, D), probs, logits, topk_idx
